# baseline (device time: 16978 ns/iter reference)
import jax
import jax.numpy as jnp
from jax import lax
from jax.experimental import pallas as pl
from jax.experimental.pallas import tpu as pltpu

_NDEV = 16
_ROWS = 512
_NCHUNK = 2
_CROWS = _ROWS // _NCHUNK

_OFFSETS = [
    (dx, dyy, dz)
    for dx in range(2)
    for dyy in range(2)
    for dz in range(4)
    if (dx, dyy, dz) != (0, 0, 0)
]


def _slot(o):
    dx, dyy, dz = o
    return dx * 8 + dyy * 4 + dz


def _inv(o):
    dx, dyy, dz = o
    return (dx, dyy, (4 - dz) % 4)


def kernel(x, dy, gamma):
    m, d = x.shape

    def body(x_hbm, dy_hbm, out_ref, xb, dyb, comm_ref,
             load_sems, send_sems, recv_sems):
        my_x = lax.axis_index("x")
        my_y = lax.axis_index("y")
        my_z = lax.axis_index("z")
        r = my_y * 4 + my_z
        off = r * _ROWS

        def target(o):
            dx, dyy, dz = o
            return (my_x ^ dx, my_y ^ dyy, lax.rem(my_z + dz, 4))

        barrier_sem = pltpu.get_barrier_semaphore()
        for o in _OFFSETS:
            pl.semaphore_signal(
                barrier_sem, inc=1,
                device_id=target(o), device_id_type=pl.DeviceIdType.MESH,
            )

        loads = []
        for c in range(_NCHUNK):
            cp_x = pltpu.make_async_copy(
                x_hbm.at[pl.ds(off + c * _CROWS, _CROWS), :],
                xb.at[c], load_sems.at[2 * c])
            cp_dy = pltpu.make_async_copy(
                dy_hbm.at[pl.ds(off + c * _CROWS, _CROWS), :],
                dyb.at[c], load_sems.at[2 * c + 1])
            cp_x.start()
            cp_dy.start()
            loads.append((cp_x, cp_dy))

        sends = []
        ones_col = jnp.ones((d, 1), jnp.float32)
        for c in range(_NCHUNK):
            cp_x, cp_dy = loads[c]
            cp_x.wait()
            cp_dy.wait()
            xv = xb[c]
            dyv = dyb[c]
            s1 = jnp.dot(xv, ones_col, preferred_element_type=jnp.float32)
            s2 = jnp.dot(xv * xv, ones_col, preferred_element_type=jnp.float32)
            mu = s1 * (1.0 / d)
            var = s2 * (1.0 / d) - mu * mu
            rstd = lax.rsqrt(var + 1e-5)
            w1 = rstd.reshape(1, _CROWS)
            w2 = jnp.concatenate(
                [(-mu * rstd).reshape(1, _CROWS),
                 jnp.ones((1, _CROWS), jnp.float32)],
                axis=0,
            )
            g1 = jnp.dot(w1, xv * dyv, preferred_element_type=jnp.float32)
            g2 = jnp.dot(w2, dyv, preferred_element_type=jnp.float32)
            comm_ref[c, 0, 0, :] = g1[0] + g2[0]
            comm_ref[c, 0, 1, :] = g2[1]

            if c == 0:
                pl.semaphore_wait(barrier_sem, len(_OFFSETS))

            for o in _OFFSETS:
                s = _slot(_inv(o))
                rdma = pltpu.make_async_remote_copy(
                    src_ref=comm_ref.at[c, 0],
                    dst_ref=comm_ref.at[c, s],
                    send_sem=send_sems.at[c, _slot(o)],
                    recv_sem=recv_sems.at[c, s],
                    device_id=target(o),
                    device_id_type=pl.DeviceIdType.MESH,
                )
                rdma.start()
                sends.append(rdma)

        for c in range(_NCHUNK):
            for o in _OFFSETS:
                s = _slot(o)
                recv = pltpu.make_async_remote_copy(
                    src_ref=comm_ref.at[c, 0],
                    dst_ref=comm_ref.at[c, s],
                    send_sem=send_sems.at[c, s],
                    recv_sem=recv_sems.at[c, s],
                    device_id=(my_x, my_y, my_z),
                    device_id_type=pl.DeviceIdType.MESH,
                )
                recv.wait_recv()
        for rdma in sends:
            rdma.wait_send()

        out_ref[:, :] = jnp.sum(
            comm_ref[:, :, :, :].reshape(_NCHUNK * _NDEV, 2, d), axis=0)

    return pl.pallas_call(
        body,
        out_shape=jax.ShapeDtypeStruct((2, d), jnp.float32),
        in_specs=[
            pl.BlockSpec(memory_space=pl.ANY),
            pl.BlockSpec(memory_space=pl.ANY),
        ],
        out_specs=pl.BlockSpec(memory_space=pltpu.VMEM),
        scratch_shapes=[
            pltpu.VMEM((_NCHUNK, _CROWS, d), jnp.float32),
            pltpu.VMEM((_NCHUNK, _CROWS, d), jnp.float32),
            pltpu.VMEM((_NCHUNK, _NDEV, 2, d), jnp.float32),
            pltpu.SemaphoreType.DMA((2 * _NCHUNK,)),
            pltpu.SemaphoreType.DMA((_NCHUNK, _NDEV)),
            pltpu.SemaphoreType.DMA((_NCHUNK, _NDEV)),
        ],
        compiler_params=pltpu.CompilerParams(collective_id=0),
    )(x, dy)


# device time: 14703 ns/iter; 1.1547x vs baseline; 1.1547x over previous
import jax
import jax.numpy as jnp
from jax import lax
from jax.experimental import pallas as pl
from jax.experimental.pallas import tpu as pltpu

_ROWS = 512
_NCHUNK = 2
_CROWS = _ROWS // _NCHUNK

_Z_OFFS = (1, 2, 3)
_XY_OFFS = ((0, 1), (1, 0), (1, 1))


def kernel(x, dy, gamma):
    m, d = x.shape

    def body(x_hbm, dy_hbm, out_ref, xb, dyb, zbuf, xybuf,
             load_sems, zsend_sems, zrecv_sems, xysend_sems, xyrecv_sems):
        my_x = lax.axis_index("x")
        my_y = lax.axis_index("y")
        my_z = lax.axis_index("z")
        r = my_y * 4 + my_z
        off = r * _ROWS

        barrier_sem = pltpu.get_barrier_semaphore()
        for dz in _Z_OFFS:
            pl.semaphore_signal(
                barrier_sem, inc=1,
                device_id=(my_x, my_y, lax.rem(my_z + dz, 4)),
                device_id_type=pl.DeviceIdType.MESH,
            )
        for dx, dyy in _XY_OFFS:
            pl.semaphore_signal(
                barrier_sem, inc=1,
                device_id=(my_x ^ dx, my_y ^ dyy, my_z),
                device_id_type=pl.DeviceIdType.MESH,
            )

        loads = []
        for c in range(_NCHUNK):
            cp_x = pltpu.make_async_copy(
                x_hbm.at[pl.ds(off + c * _CROWS, _CROWS), :],
                xb.at[c], load_sems.at[2 * c])
            cp_dy = pltpu.make_async_copy(
                dy_hbm.at[pl.ds(off + c * _CROWS, _CROWS), :],
                dyb.at[c], load_sems.at[2 * c + 1])
            cp_x.start()
            cp_dy.start()
            loads.append((cp_x, cp_dy))

        ones_col = jnp.ones((d, 1), jnp.float32)
        pg = None
        pb = None
        for c in range(_NCHUNK):
            cp_x, cp_dy = loads[c]
            cp_x.wait()
            cp_dy.wait()
            xv = xb[c]
            dyv = dyb[c]
            s1 = jnp.dot(xv, ones_col, preferred_element_type=jnp.float32)
            s2 = jnp.dot(xv * xv, ones_col, preferred_element_type=jnp.float32)
            mu = s1 * (1.0 / d)
            var = s2 * (1.0 / d) - mu * mu
            rstd = lax.rsqrt(var + 1e-5)
            w1 = rstd.reshape(1, _CROWS)
            w2 = jnp.concatenate(
                [(-mu * rstd).reshape(1, _CROWS),
                 jnp.ones((1, _CROWS), jnp.float32)],
                axis=0,
            )
            g1 = jnp.dot(w1, xv * dyv, preferred_element_type=jnp.float32)
            g2 = jnp.dot(w2, dyv, preferred_element_type=jnp.float32)
            dgamma = g1[0] + g2[0]
            dbeta = g2[1]
            pg = dgamma if pg is None else pg + dgamma
            pb = dbeta if pb is None else pb + dbeta
        zbuf[0, 0, :] = pg
        zbuf[0, 1, :] = pb

        pl.semaphore_wait(barrier_sem, len(_Z_OFFS) + len(_XY_OFFS))

        zsends = []
        for dz in _Z_OFFS:
            s = (4 - dz) % 4
            rdma = pltpu.make_async_remote_copy(
                src_ref=zbuf.at[0],
                dst_ref=zbuf.at[s],
                send_sem=zsend_sems.at[dz],
                recv_sem=zrecv_sems.at[s],
                device_id=(my_x, my_y, lax.rem(my_z + dz, 4)),
                device_id_type=pl.DeviceIdType.MESH,
            )
            rdma.start()
            zsends.append(rdma)
        for dz in _Z_OFFS:
            recv = pltpu.make_async_remote_copy(
                src_ref=zbuf.at[0],
                dst_ref=zbuf.at[dz],
                send_sem=zsend_sems.at[dz],
                recv_sem=zrecv_sems.at[dz],
                device_id=(my_x, my_y, my_z),
                device_id_type=pl.DeviceIdType.MESH,
            )
            recv.wait_recv()

        xybuf[0, :, :] = (
            zbuf[0, :, :] + zbuf[1, :, :] + zbuf[2, :, :] + zbuf[3, :, :]
        )
        xysends = []
        for dx, dyy in _XY_OFFS:
            s = dx * 2 + dyy
            rdma = pltpu.make_async_remote_copy(
                src_ref=xybuf.at[0],
                dst_ref=xybuf.at[s],
                send_sem=xysend_sems.at[s],
                recv_sem=xyrecv_sems.at[s],
                device_id=(my_x ^ dx, my_y ^ dyy, my_z),
                device_id_type=pl.DeviceIdType.MESH,
            )
            rdma.start()
            xysends.append(rdma)
        for dx, dyy in _XY_OFFS:
            s = dx * 2 + dyy
            recv = pltpu.make_async_remote_copy(
                src_ref=xybuf.at[0],
                dst_ref=xybuf.at[s],
                send_sem=xysend_sems.at[s],
                recv_sem=xyrecv_sems.at[s],
                device_id=(my_x, my_y, my_z),
                device_id_type=pl.DeviceIdType.MESH,
            )
            recv.wait_recv()
        for rdma in zsends:
            rdma.wait_send()
        for rdma in xysends:
            rdma.wait_send()

        out_ref[:, :] = (
            xybuf[0, :, :] + xybuf[1, :, :] + xybuf[2, :, :] + xybuf[3, :, :]
        )

    return pl.pallas_call(
        body,
        out_shape=jax.ShapeDtypeStruct((2, d), jnp.float32),
        in_specs=[
            pl.BlockSpec(memory_space=pl.ANY),
            pl.BlockSpec(memory_space=pl.ANY),
        ],
        out_specs=pl.BlockSpec(memory_space=pltpu.VMEM),
        scratch_shapes=[
            pltpu.VMEM((_NCHUNK, _CROWS, d), jnp.float32),
            pltpu.VMEM((_NCHUNK, _CROWS, d), jnp.float32),
            pltpu.VMEM((4, 2, d), jnp.float32),
            pltpu.VMEM((4, 2, d), jnp.float32),
            pltpu.SemaphoreType.DMA((2 * _NCHUNK,)),
            pltpu.SemaphoreType.DMA((4,)),
            pltpu.SemaphoreType.DMA((4,)),
            pltpu.SemaphoreType.DMA((4,)),
            pltpu.SemaphoreType.DMA((4,)),
        ],
        compiler_params=pltpu.CompilerParams(collective_id=0),
    )(x, dy)
